# baseline (speedup 1.0000x reference)
import functools

import jax
import jax.numpy as jnp
from jax import lax
from jax.experimental import pallas as pl
from jax.experimental.pallas import tpu as pltpu
from jax.experimental.pallas import tpu_sc as plsc


def _dist_argmin_kernel(x2_ref, e2_ref, x_ref, emb_ref, ind_ref, *, kc):
    nt = x_ref.shape[1]
    k = emb_ref.shape[1]
    xb = (x_ref[0] * -2.0).astype(jnp.bfloat16)
    x2col = x2_ref[0, 0][:, None]

    big = jnp.float32(jnp.inf)
    m = jnp.full((nt, kc), big, dtype=jnp.float32)
    idx = jnp.zeros((nt, kc), dtype=jnp.int32)
    for kt in range(k // kc):
        eb = emb_ref[0, pl.ds(kt * kc, kc), :]
        cr = lax.dot_general(xb, eb, (((1,), (1,)), ((), ())),
                             preferred_element_type=jnp.float32)
        t1 = x2col + e2_ref[0, 0, pl.ds(kt * kc, kc)][None, :]
        d2 = t1 + cr
        s = jnp.sqrt(jnp.maximum(d2, 0.0))
        kio = lax.broadcasted_iota(jnp.int32, (nt, kc), 1) + (kt * kc)
        better = s < m
        m = jnp.where(better, s, m)
        idx = jnp.where(better, kio, idx)
    mrow = jnp.min(m, axis=1, keepdims=True)
    idxw = jnp.where(m == mrow, idx, jnp.int32(2 * k))
    ind_ref[0, 0] = jnp.min(idxw, axis=1)


def _dist_argmin(x2, e2, flatten, key_embed, nt, kc):
    h, n, d = flatten.shape
    k = key_embed.shape[1]
    nb = n // nt
    x2r = x2.reshape(h * nb, 1, nt)
    e2r = e2.reshape(h, 1, k)
    out = pl.pallas_call(
        functools.partial(_dist_argmin_kernel, kc=kc),
        grid=(h, nb),
        in_specs=[
            pl.BlockSpec((1, 1, nt), lambda hh, i: (hh * nb + i, 0, 0)),
            pl.BlockSpec((1, 1, k), lambda hh, i: (hh, 0, 0)),
            pl.BlockSpec((1, nt, d), lambda hh, i: (hh, i, 0)),
            pl.BlockSpec((1, k, d), lambda hh, i: (hh, 0, 0)),
        ],
        out_specs=pl.BlockSpec((1, 1, nt), lambda hh, i: (hh * nb + i, 0, 0)),
        out_shape=jax.ShapeDtypeStruct((h * nb, 1, nt), jnp.int32),
        compiler_params=pltpu.CompilerParams(
            dimension_semantics=("arbitrary", "arbitrary"),
        ),
    )(x2r, e2r, flatten, key_embed.astype(jnp.bfloat16))
    return out.reshape(h, n)


def _gather_body(table_hbm, idx_hbm, out_hbm, idx_v, rows_v, sem):
    nc = 2
    wid = lax.axis_index("s") * nc + lax.axis_index("c")
    bpw = idx_v.shape[0]
    ch = rows_v.shape[0]
    base = wid * bpw
    pltpu.sync_copy(idx_hbm.at[pl.ds(base, bpw)], idx_v)
    for ci in range(bpw // ch):
        pltpu.async_copy(
            table_hbm.at[idx_v.at[pl.ds(ci * ch, ch)]], rows_v, sem).wait()
        pltpu.sync_copy(rows_v, out_hbm.at[pl.ds(base + ci * ch, ch)])


def _gather_rows(table, gidx, bpw, ch):
    rows, d = table.shape
    nidx = gidx.shape[0]
    kfn = pl.kernel(
        _gather_body,
        out_type=jax.ShapeDtypeStruct((nidx, d), jnp.float32),
        mesh=plsc.VectorSubcoreMesh(core_axis_name="c", subcore_axis_name="s"),
        scratch_types=[
            pltpu.VMEM((bpw,), jnp.int32),
            pltpu.VMEM((ch, d), jnp.float32),
            pltpu.SemaphoreType.DMA,
        ],
    )
    return kfn(table, gidx)


def kernel(x, key_embed, key_optim):
    x = x.astype(jnp.float32)
    b, t, _ = x.shape
    h, k, d = key_embed.shape
    n = b * t
    xh = x.reshape(b, t, h, d).transpose(2, 0, 1, 3)
    flatten = xh.reshape(h, n, d)
    x2 = jnp.sum(x.reshape(b, t, h, d) ** 2, -1).transpose(2, 0, 1).reshape(h, n)
    e2 = jnp.sum(key_embed * key_embed, axis=-1)

    inds2 = _dist_argmin(x2, e2, flatten, key_embed, nt=256, kc=512)

    offs = (jnp.arange(h, dtype=jnp.int32) * k)[:, None]
    gidx = (inds2 + offs).T.reshape(-1)
    table = key_embed.reshape(h * k, d)
    nw = 32
    rows = _gather_rows(table, gidx, bpw=(n * h) // nw, ch=288)
    quantized = rows.reshape(b, t, h * d)
    inds = inds2.T.reshape(b, t, h)
    return (quantized, inds)


# speedup vs baseline: 1.1568x; 1.1568x over previous
import functools

import jax
import jax.numpy as jnp
from jax import lax
from jax.experimental import pallas as pl
from jax.experimental.pallas import tpu as pltpu
from jax.experimental.pallas import tpu_sc as plsc


def _dist_argmin_kernel(x2_ref, e2_ref, x_ref, emb_ref, ind_ref, *, kc):
    nt, d = x_ref.shape
    k = emb_ref.shape[1]
    xb = (x_ref[...] * -2.0).astype(jnp.bfloat16)
    x2col = x2_ref[0, 0][:, None]

    big = jnp.float32(jnp.inf)
    lane_f = lax.broadcasted_iota(jnp.int32, (nt, kc), 1).astype(jnp.float32)
    mrow = jnp.full((nt, 1), big, dtype=jnp.float32)
    lrow = jnp.zeros((nt, 1), dtype=jnp.float32)
    crow = jnp.zeros((nt, 1), dtype=jnp.float32)
    for kt in range(k // kc):
        eb = emb_ref[0, pl.ds(kt * kc, kc), :]
        cr = lax.dot_general(xb, eb, (((1,), (1,)), ((), ())),
                             preferred_element_type=jnp.float32)
        t1 = x2col + e2_ref[0, 0, pl.ds(kt * kc, kc)][None, :]
        c = jnp.maximum(t1 + cr, 0.0)
        s = jnp.where(c == 0.0, 0.0, c * lax.rsqrt(c))
        cmin = jnp.min(s, axis=1, keepdims=True)
        lmin = jnp.min(jnp.where(s == cmin, lane_f, big), axis=1, keepdims=True)
        upd = cmin < mrow
        mrow = jnp.where(upd, cmin, mrow)
        lrow = jnp.where(upd, lmin, lrow)
        crow = jnp.where(upd, jnp.float32(kt), crow)
    ind = crow * jnp.float32(kc) + lrow
    ind_ref[0, 0] = ind[:, 0].astype(jnp.int32)


def _dist_argmin(x2, e2, x2d, key_embed_bf16, nt, kc):
    n, hd = x2d.shape
    h, k, d = key_embed_bf16.shape
    nb = n // nt
    x2r = x2.reshape(h * nb, 1, nt)
    e2r = e2.reshape(h, 1, k)
    out = pl.pallas_call(
        functools.partial(_dist_argmin_kernel, kc=kc),
        grid=(h, nb),
        in_specs=[
            pl.BlockSpec((1, 1, nt), lambda hh, i: (hh * nb + i, 0, 0)),
            pl.BlockSpec((1, 1, k), lambda hh, i: (hh, 0, 0)),
            pl.BlockSpec((nt, d), lambda hh, i: (i, hh)),
            pl.BlockSpec((1, k, d), lambda hh, i: (hh, 0, 0)),
        ],
        out_specs=pl.BlockSpec((1, 1, nt), lambda hh, i: (hh * nb + i, 0, 0)),
        out_shape=jax.ShapeDtypeStruct((h * nb, 1, nt), jnp.int32),
        compiler_params=pltpu.CompilerParams(
            dimension_semantics=("arbitrary", "arbitrary"),
        ),
    )(x2r, e2r, x2d, key_embed_bf16)
    return out.reshape(h, n)


def _gather_body(table_hbm, idx_hbm, out_hbm, idx_v, rows_v, sem):
    nc = 2
    wid = lax.axis_index("s") * nc + lax.axis_index("c")
    bpw = idx_v.shape[0]
    ch = rows_v.shape[0]
    base = wid * bpw
    pltpu.sync_copy(idx_hbm.at[pl.ds(base, bpw)], idx_v)
    for ci in range(bpw // ch):
        pltpu.async_copy(
            table_hbm.at[idx_v.at[pl.ds(ci * ch, ch)]], rows_v, sem).wait()
        pltpu.sync_copy(rows_v, out_hbm.at[pl.ds(base + ci * ch, ch)])


def _gather_rows(table, gidx, bpw, ch):
    rows, d = table.shape
    nidx = gidx.shape[0]
    kfn = pl.kernel(
        _gather_body,
        out_type=jax.ShapeDtypeStruct((nidx, d), jnp.float32),
        mesh=plsc.VectorSubcoreMesh(core_axis_name="c", subcore_axis_name="s"),
        scratch_types=[
            pltpu.VMEM((bpw,), jnp.int32),
            pltpu.VMEM((ch, d), jnp.float32),
            pltpu.SemaphoreType.DMA,
        ],
    )
    return kfn(table, gidx)


def kernel(x, key_embed, key_optim):
    x = x.astype(jnp.float32)
    b, t, _ = x.shape
    h, k, d = key_embed.shape
    n = b * t
    x2 = jnp.sum(x.reshape(b, t, h, d) ** 2, -1).transpose(2, 0, 1).reshape(h, n)
    e2 = jnp.sum(key_embed * key_embed, axis=-1)
    x2d = x.reshape(n, h * d)

    inds2 = _dist_argmin(x2, e2, x2d, key_embed.astype(jnp.bfloat16),
                         nt=512, kc=512)

    offs = (jnp.arange(h, dtype=jnp.int32) * k)[:, None]
    gidx = (inds2 + offs).T.reshape(-1)
    table = key_embed.reshape(h * k, d)
    nw = 32
    rows = _gather_rows(table, gidx, bpw=(n * h) // nw, ch=288)
    quantized = rows.reshape(b, t, h * d)
    inds = inds2.T.reshape(b, t, h)
    return (quantized, inds)


# speedup vs baseline: 1.3047x; 1.1278x over previous
import functools

import jax
import jax.numpy as jnp
from jax import lax
from jax.experimental import pallas as pl
from jax.experimental.pallas import tpu as pltpu
from jax.experimental.pallas import tpu_sc as plsc


def _dist_argmin_kernel(x2_ref, e2_ref, x_ref, emb_ref, ind_ref, *, kc):
    nt, d = x_ref.shape
    k = emb_ref.shape[1]
    xb = (x_ref[...] * -2.0).astype(jnp.bfloat16)
    x2col = x2_ref[0, 0][:, None]

    big = jnp.float32(jnp.inf)
    lane_f = lax.broadcasted_iota(jnp.int32, (nt, kc), 1).astype(jnp.float32)
    mrow = jnp.full((nt, 1), big, dtype=jnp.float32)
    lrow = jnp.zeros((nt, 1), dtype=jnp.float32)
    crow = jnp.zeros((nt, 1), dtype=jnp.float32)
    for kt in range(k // kc):
        eb = emb_ref[0, pl.ds(kt * kc, kc), :]
        cr = lax.dot_general(xb, eb, (((1,), (1,)), ((), ())),
                             preferred_element_type=jnp.float32)
        t1 = x2col + e2_ref[0, 0, pl.ds(kt * kc, kc)][None, :]
        c = jnp.maximum(t1 + cr, 0.0)
        s = jnp.where(c == 0.0, 0.0, c * lax.rsqrt(c))
        cmin = jnp.min(s, axis=1, keepdims=True)
        lmin = jnp.min(jnp.where(s == cmin, lane_f, big), axis=1, keepdims=True)
        upd = cmin < mrow
        mrow = jnp.where(upd, cmin, mrow)
        lrow = jnp.where(upd, lmin, lrow)
        crow = jnp.where(upd, jnp.float32(kt), crow)
    ind = crow * jnp.float32(kc) + lrow
    ind_ref[0, 0] = ind[:, 0].astype(jnp.int32)


def _dist_argmin(x2, e2, x2d, key_embed_bf16, nt, kc):
    n, hd = x2d.shape
    h, k, d = key_embed_bf16.shape
    nb = n // nt
    x2r = x2.reshape(h * nb, 1, nt)
    e2r = e2.reshape(h, 1, k)
    out = pl.pallas_call(
        functools.partial(_dist_argmin_kernel, kc=kc),
        grid=(h, nb),
        in_specs=[
            pl.BlockSpec((1, 1, nt), lambda hh, i: (hh * nb + i, 0, 0)),
            pl.BlockSpec((1, 1, k), lambda hh, i: (hh, 0, 0)),
            pl.BlockSpec((nt, d), lambda hh, i: (i, hh)),
            pl.BlockSpec((1, k, d), lambda hh, i: (hh, 0, 0)),
        ],
        out_specs=pl.BlockSpec((1, 1, nt), lambda hh, i: (hh * nb + i, 0, 0)),
        out_shape=jax.ShapeDtypeStruct((h * nb, 1, nt), jnp.int32),
        compiler_params=pltpu.CompilerParams(
            dimension_semantics=("arbitrary", "arbitrary"),
        ),
    )(x2r, e2r, x2d, key_embed_bf16)
    return out.reshape(h, n)


def _gather_body(table_hbm, idx_hbm, out_hbm, idx_v, rows_v, sem):
    nc = 2
    wid = lax.axis_index("s") * nc + lax.axis_index("c")
    bpw = idx_v.shape[0]
    ch = rows_v.shape[0]
    base = wid * bpw
    pltpu.sync_copy(idx_hbm.at[pl.ds(base, bpw)], idx_v)
    for ci in range(bpw // ch):
        pltpu.async_copy(
            table_hbm.at[idx_v.at[pl.ds(ci * ch, ch)]], rows_v, sem).wait()
        pltpu.sync_copy(rows_v, out_hbm.at[pl.ds(base + ci * ch, ch)])


def _gather_rows(table, gidx, bpw, ch):
    rows, d = table.shape
    nidx = gidx.shape[0]
    kfn = pl.kernel(
        _gather_body,
        out_type=jax.ShapeDtypeStruct((nidx, d), jnp.float32),
        mesh=plsc.VectorSubcoreMesh(core_axis_name="c", subcore_axis_name="s"),
        scratch_types=[
            pltpu.VMEM((bpw,), jnp.int32),
            pltpu.VMEM((ch, d), jnp.float32),
            pltpu.SemaphoreType.DMA,
        ],
    )
    return kfn(table, gidx)


def kernel(x, key_embed, key_optim):
    x = x.astype(jnp.float32)
    b, t, _ = x.shape
    h, k, d = key_embed.shape
    n = b * t
    x2 = jnp.sum(x.reshape(b, t, h, d) ** 2, -1).transpose(2, 0, 1).reshape(h, n)
    e2 = jnp.sum(key_embed * key_embed, axis=-1)
    x2d = x.reshape(n, h * d)

    inds2 = _dist_argmin(x2, e2, x2d, key_embed.astype(jnp.bfloat16),
                         nt=1024, kc=512)

    offs = (jnp.arange(h, dtype=jnp.int32) * k)[:, None]
    gidx = (inds2 + offs).T.reshape(-1)
    table = key_embed.reshape(h * k, d)
    nw = 32
    rows = _gather_rows(table, gidx, bpw=(n * h) // nw, ch=288)
    quantized = rows.reshape(b, t, h * d)
    inds = inds2.T.reshape(b, t, h)
    return (quantized, inds)


# speedup vs baseline: 1.3430x; 1.0294x over previous
import functools

import jax
import jax.numpy as jnp
from jax import lax
from jax.experimental import pallas as pl
from jax.experimental.pallas import tpu as pltpu
from jax.experimental.pallas import tpu_sc as plsc


def _dist_argmin_kernel(x2_ref, e2_ref, x_ref, emb_ref, ind_ref, *, kc):
    nt, d = x_ref.shape
    k = emb_ref.shape[1]
    xb = (x_ref[...] * -2.0).astype(jnp.bfloat16)
    x2col = x2_ref[0, 0][:, None]

    big = jnp.float32(jnp.inf)
    lane_f = lax.broadcasted_iota(jnp.int32, (nt, kc), 1).astype(jnp.float32)
    mrow = jnp.full((nt, 1), big, dtype=jnp.float32)
    lrow = jnp.zeros((nt, 1), dtype=jnp.float32)
    crow = jnp.zeros((nt, 1), dtype=jnp.float32)
    for kt in range(k // kc):
        eb = emb_ref[0, pl.ds(kt * kc, kc), :]
        cr = lax.dot_general(xb, eb, (((1,), (1,)), ((), ())),
                             preferred_element_type=jnp.float32)
        t1 = x2col + e2_ref[0, 0, pl.ds(kt * kc, kc)][None, :]
        c = jnp.maximum(t1 + cr, 0.0)
        s = jnp.where(c == 0.0, 0.0, c * lax.rsqrt(c))
        cmin = jnp.min(s, axis=1, keepdims=True)
        lmin = jnp.min(jnp.where(s == cmin, lane_f, big), axis=1, keepdims=True)
        upd = cmin < mrow
        mrow = jnp.where(upd, cmin, mrow)
        lrow = jnp.where(upd, lmin, lrow)
        crow = jnp.where(upd, jnp.float32(kt), crow)
    ind = crow * jnp.float32(kc) + lrow
    ind_ref[0, 0] = ind[:, 0].astype(jnp.int32)


def _dist_argmin(x2, e2, x2d, key_embed_bf16, nt, kc):
    n, hd = x2d.shape
    h, k, d = key_embed_bf16.shape
    nb = n // nt
    x2r = x2.reshape(h * nb, 1, nt)
    e2r = e2.reshape(h, 1, k)
    out = pl.pallas_call(
        functools.partial(_dist_argmin_kernel, kc=kc),
        grid=(h, nb),
        in_specs=[
            pl.BlockSpec((1, 1, nt), lambda hh, i: (hh * nb + i, 0, 0)),
            pl.BlockSpec((1, 1, k), lambda hh, i: (hh, 0, 0)),
            pl.BlockSpec((nt, d), lambda hh, i: (i, hh)),
            pl.BlockSpec((1, k, d), lambda hh, i: (hh, 0, 0)),
        ],
        out_specs=pl.BlockSpec((1, 1, nt), lambda hh, i: (hh * nb + i, 0, 0)),
        out_shape=jax.ShapeDtypeStruct((h * nb, 1, nt), jnp.int32),
        compiler_params=pltpu.CompilerParams(
            dimension_semantics=("arbitrary", "arbitrary"),
        ),
    )(x2r, e2r, x2d, key_embed_bf16)
    return out.reshape(h, n)


def _gather_body(table_hbm, idx_hbm, out_hbm, idx_v, rows_v, sem):
    nc = 2
    wid = lax.axis_index("s") * nc + lax.axis_index("c")
    bpw = idx_v.shape[0]
    ch = rows_v.shape[0]
    base = wid * bpw
    pltpu.sync_copy(idx_hbm.at[pl.ds(base, bpw)], idx_v)
    for ci in range(bpw // ch):
        pltpu.async_copy(
            table_hbm.at[idx_v.at[pl.ds(ci * ch, ch)]], rows_v, sem).wait()
        pltpu.sync_copy(rows_v, out_hbm.at[pl.ds(base + ci * ch, ch)])


def _gather_rows(table, gidx, bpw, ch):
    rows, d = table.shape
    nidx = gidx.shape[0]
    kfn = pl.kernel(
        _gather_body,
        out_type=jax.ShapeDtypeStruct((nidx, d), jnp.float32),
        mesh=plsc.VectorSubcoreMesh(core_axis_name="c", subcore_axis_name="s"),
        scratch_types=[
            pltpu.VMEM((bpw,), jnp.int32),
            pltpu.VMEM((ch, d), jnp.float32),
            pltpu.SemaphoreType.DMA,
        ],
    )
    return kfn(table, gidx)


def kernel(x, key_embed, key_optim):
    x = x.astype(jnp.float32)
    b, t, _ = x.shape
    h, k, d = key_embed.shape
    n = b * t
    x2 = jnp.sum(x.reshape(b, t, h, d) ** 2, -1).transpose(2, 0, 1).reshape(h, n)
    e2 = jnp.sum(key_embed * key_embed, axis=-1)
    x2d = x.reshape(n, h * d)

    inds2 = _dist_argmin(x2, e2, x2d, key_embed.astype(jnp.bfloat16),
                         nt=1536, kc=512)

    offs = (jnp.arange(h, dtype=jnp.int32) * k)[:, None]
    gidx = (inds2 + offs).T.reshape(-1)
    table = key_embed.reshape(h * k, d)
    nw = 32
    rows = _gather_rows(table, gidx, bpw=(n * h) // nw, ch=288)
    quantized = rows.reshape(b, t, h * d)
    inds = inds2.T.reshape(b, t, h)
    return (quantized, inds)
